# baseline (speedup 1.0000x reference)
import functools

import jax
import jax.numpy as jnp
from jax import lax
from jax.experimental import pallas as pl
from jax.experimental.pallas import tpu as pltpu
from jax.experimental.pallas import tpu_sc as plsc

BATCH = 16384
D_PER = 64
D_OUT = 2 * D_PER

NUM_CORES = 2
NUM_SUBCORES = 16
NUM_WORKERS = NUM_CORES * NUM_SUBCORES
B_PER_W = BATCH // NUM_WORKERS
CHUNK = 128
N_CHUNKS = B_PER_W // CHUNK
HALF = B_PER_W // 2
CHUNKS_PER_HALF = HALF // CHUNK

_mesh = plsc.VectorSubcoreMesh(core_axis_name="c", subcore_axis_name="s")


@functools.partial(
    pl.kernel,
    mesh=_mesh,
    compiler_params=pltpu.CompilerParams(use_tc_tiling_on_sc=False),
    out_type=jax.ShapeDtypeStruct((BATCH, D_OUT), jnp.float32),
    scratch_types=[
        pltpu.VMEM((N_CHUNKS, CHUNK), jnp.int32),
        pltpu.VMEM((N_CHUNKS, CHUNK), jnp.int32),
        pltpu.VMEM((HALF, D_PER), jnp.float32),
        pltpu.VMEM((HALF, D_PER), jnp.float32),
        pltpu.VMEM_SHARED((NUM_SUBCORES * B_PER_W, D_OUT), jnp.float32),
        pltpu.SemaphoreType.DMA,
    ],
)
def _mle_kernel(idx0_hbm, idx1_hbm, w0_hbm, w1_hbm, out_hbm,
                idx0_v, idx1_v, rows0_v, rows1_v, comb_sh, sem):
    cid = lax.axis_index("c")
    sid = lax.axis_index("s")
    wid = sid * NUM_CORES + cid
    base = wid * B_PER_W
    slab = sid * B_PER_W

    pltpu.sync_copy(idx0_hbm.at[wid], idx0_v)
    pltpu.sync_copy(idx1_hbm.at[wid], idx1_v)

    for h in range(2):
        copies = []
        for cc in range(CHUNKS_PER_HALF):
            c = h * CHUNKS_PER_HALF + cc
            rows = pl.ds(cc * CHUNK, CHUNK)
            copies.append(pltpu.async_copy(
                w0_hbm.at[idx0_v.at[c]], rows0_v.at[rows], sem))
            copies.append(pltpu.async_copy(
                w1_hbm.at[idx1_v.at[c]], rows1_v.at[rows], sem))
        for cp in copies:
            cp.wait()

        slab_rows = pl.ds(slab + h * HALF, HALF)
        pltpu.sync_copy(rows0_v, comb_sh.at[slab_rows, pl.ds(0, D_PER)])
        pltpu.sync_copy(rows1_v, comb_sh.at[slab_rows, pl.ds(D_PER, D_PER)])
        pltpu.sync_copy(comb_sh.at[slab_rows],
                        out_hbm.at[pl.ds(base + h * HALF, HALF)])


def kernel(y, W0, W1):
    idx0 = y[:, 0].astype(jnp.int32).reshape(NUM_WORKERS, N_CHUNKS, CHUNK)
    idx1 = y[:, 1].astype(jnp.int32).reshape(NUM_WORKERS, N_CHUNKS, CHUNK)
    return _mle_kernel(idx0, idx1, W0, W1)


# speedup vs baseline: 1.0307x; 1.0307x over previous
import functools

import jax
import jax.numpy as jnp
from jax import lax
from jax.experimental import pallas as pl
from jax.experimental.pallas import tpu as pltpu
from jax.experimental.pallas import tpu_sc as plsc

BATCH = 16384
D_PER = 64
D_OUT = 2 * D_PER

NUM_CORES = 2
NUM_SUBCORES = 16
NUM_WORKERS = NUM_CORES * NUM_SUBCORES
B_PER_W = BATCH // NUM_WORKERS
CHUNK = 128
N_CHUNKS = B_PER_W // CHUNK

_mesh = plsc.VectorSubcoreMesh(core_axis_name="c", subcore_axis_name="s")


@functools.partial(
    pl.kernel,
    mesh=_mesh,
    compiler_params=pltpu.CompilerParams(use_tc_tiling_on_sc=False),
    out_type=jax.ShapeDtypeStruct((BATCH, D_OUT), jnp.float32),
    scratch_types=[
        pltpu.VMEM((N_CHUNKS, CHUNK), jnp.int32),
        pltpu.VMEM((N_CHUNKS, CHUNK), jnp.int32),
        pltpu.VMEM((B_PER_W, D_PER), jnp.float32),
        pltpu.VMEM((B_PER_W, D_PER), jnp.float32),
        pltpu.SemaphoreType.DMA,
    ],
)
def _mle_kernel(idx0_hbm, idx1_hbm, w0_hbm, w1_hbm, out_hbm,
                idx0_v, idx1_v, rows0_v, rows1_v, sem):
    cid = lax.axis_index("c")
    sid = lax.axis_index("s")
    wid = sid * NUM_CORES + cid
    base = wid * B_PER_W

    pltpu.sync_copy(idx0_hbm.at[wid], idx0_v)
    pltpu.sync_copy(idx1_hbm.at[wid], idx1_v)

    copies = []
    for c in range(N_CHUNKS):
        rows = pl.ds(c * CHUNK, CHUNK)
        copies.append(pltpu.async_copy(
            w0_hbm.at[idx0_v.at[c]], rows0_v.at[rows], sem))
        copies.append(pltpu.async_copy(
            w1_hbm.at[idx1_v.at[c]], rows1_v.at[rows], sem))
    for cp in copies:
        cp.wait()

    out_rows = pl.ds(base, B_PER_W)
    pltpu.sync_copy(rows0_v, out_hbm.at[out_rows, pl.ds(0, D_PER)])
    pltpu.sync_copy(rows1_v, out_hbm.at[out_rows, pl.ds(D_PER, D_PER)])


def kernel(y, W0, W1):
    idx0 = y[:, 0].astype(jnp.int32).reshape(NUM_WORKERS, N_CHUNKS, CHUNK)
    idx1 = y[:, 1].astype(jnp.int32).reshape(NUM_WORKERS, N_CHUNKS, CHUNK)
    return _mle_kernel(idx0, idx1, W0, W1)
